# baseline (speedup 1.0000x reference)
import functools

import jax
import jax.numpy as jnp
from jax import lax
from jax.experimental import pallas as pl
from jax.experimental.pallas import tpu as pltpu
from jax.experimental.pallas import tpu_sc as plsc

D_MODEL = 1024
BATCH = 16384


@functools.partial(jax.jit, static_argnames=())
def _sc_embed(table, idx):
    info = plsc.get_sparse_core_info()
    nc, ns = info.num_cores, info.num_subcores
    nw = nc * ns
    b_per_w = BATCH // nw
    ch = 32
    nbuf = 3
    n_chunks = b_per_w // ch

    mesh = plsc.VectorSubcoreMesh(core_axis_name="c", subcore_axis_name="s")

    @functools.partial(
        pl.kernel,
        mesh=mesh,
        out_type=jax.ShapeDtypeStruct((BATCH, D_MODEL), jnp.float32),
        scratch_types=[
            pltpu.VMEM((b_per_w,), jnp.int32),
            pltpu.VMEM((nbuf, ch, D_MODEL), jnp.float32),
            pltpu.SemaphoreType.DMA,
            pltpu.SemaphoreType.DMA,
        ],
    )
    def k(table_hbm, idx_hbm, out_hbm, idx_v, rows_v, gsem, ssem):
        wid = lax.axis_index("s") * nc + lax.axis_index("c")
        base = wid * b_per_w
        pltpu.sync_copy(idx_hbm.at[pl.ds(base, b_per_w)], idx_v)

        def start_gather(g):
            return pltpu.async_copy(
                table_hbm.at[idx_v.at[pl.ds(g * ch, ch)]],
                rows_v.at[g % nbuf], gsem)

        ga = [None] * n_chunks
        sc = [None] * n_chunks
        for g in range(min(nbuf - 1, n_chunks)):
            ga[g] = start_gather(g)
        for g in range(n_chunks):
            nxt = g + nbuf - 1
            if nxt < n_chunks:
                if g >= 1:
                    sc[g - 1].wait()
                ga[nxt] = start_gather(nxt)
            ga[g].wait()
            sc[g] = pltpu.async_copy(
                rows_v.at[g % nbuf], out_hbm.at[pl.ds(base + g * ch, ch)], ssem)
        for g in range(max(0, n_chunks - nbuf), n_chunks):
            sc[g].wait()

    return k(table, idx)


def kernel(weeks, table):
    out = _sc_embed(table, weeks.astype(jnp.int32))
    return out[:, None, :]


# speedup vs baseline: 1.8989x; 1.8989x over previous
import functools

import jax
import jax.numpy as jnp
from jax import lax
from jax.experimental import pallas as pl
from jax.experimental.pallas import tpu as pltpu
from jax.experimental.pallas import tpu_sc as plsc

D_MODEL = 1024
BATCH = 16384


@functools.partial(jax.jit, static_argnames=())
def _sc_embed(table, idx):
    info = plsc.get_sparse_core_info()
    nc, ns = info.num_cores, info.num_subcores
    nw = nc * ns
    b_per_w = BATCH // nw
    ch = 32
    nbuf = 3
    n_chunks = b_per_w // ch

    mesh = plsc.VectorSubcoreMesh(core_axis_name="c", subcore_axis_name="s")

    @functools.partial(
        pl.kernel,
        mesh=mesh,
        out_type=jax.ShapeDtypeStruct((BATCH, D_MODEL), jnp.float32),
        scratch_types=[
            pltpu.VMEM((b_per_w,), jnp.int32),
            pltpu.VMEM((nbuf, ch, D_MODEL), jnp.float32),
            pltpu.SemaphoreType.DMA,
            pltpu.SemaphoreType.DMA,
        ],
    )
    def k(table_hbm, idx_hbm, out_hbm, idx_v, rows_v, gsem, ssem):
        wid = lax.axis_index("s") * nc + lax.axis_index("c")
        base = wid * b_per_w
        pltpu.sync_copy(idx_hbm.at[pl.ds(base, b_per_w)], idx_v)
        off = jnp.broadcast_to((wid * 13).astype(jnp.int32), (16,))
        for j in range(b_per_w // 16):
            sl = pl.ds(j * 16, 16)
            idx_v[sl] = idx_v[sl] + off

        def start_gather(g):
            return pltpu.async_copy(
                table_hbm.at[idx_v.at[pl.ds(g * ch, ch)]],
                rows_v.at[g % nbuf], gsem)

        ga = [None] * n_chunks
        sc = [None] * n_chunks
        for g in range(min(nbuf - 1, n_chunks)):
            ga[g] = start_gather(g)
        for g in range(n_chunks):
            nxt = g + nbuf - 1
            if nxt < n_chunks:
                if g >= 1:
                    sc[g - 1].wait()
                ga[nxt] = start_gather(nxt)
            ga[g].wait()
            sc[g] = pltpu.async_copy(
                rows_v.at[g % nbuf], out_hbm.at[pl.ds(base + g * ch, ch)], ssem)
        for g in range(max(0, n_chunks - nbuf), n_chunks):
            sc[g].wait()

    return k(table, idx)


def kernel(weeks, table):
    rep = jnp.tile(table, (32, 1))
    out = _sc_embed(rep, weeks.astype(jnp.int32))
    return out[:, None, :]
